# Initial kernel scaffold; baseline (speedup 1.0000x reference)
import jax
import jax.numpy as jnp
from jax.experimental import pallas as pl


def kernel(x, table, pos_table, gamma, beta):
    raise NotImplementedError("write your pallas kernel here")


# speedup vs baseline: 1.4354x; 1.4354x over previous
import functools

import jax
import jax.numpy as jnp
import numpy as np
from jax import lax
from jax.experimental import pallas as pl
from jax.experimental.pallas import tpu as pltpu
from jax.experimental.pallas import tpu_sc as plsc

V = 100000
H = 128
B = 1024
T = 200
EPS = 1e-5

NC = 2
NS = 16
NW = NC * NS
NROWS = B * T
RPW = NROWS // NW
SUB = 100
CHUNK = 200
NCHUNK = RPW // CHUNK
IDX_ROWS = RPW // SUB
HL = H // 16

_mesh = plsc.VectorSubcoreMesh(core_axis_name="c", subcore_axis_name="s")


_GDN = lax.GatherDimensionNumbers(
    offset_dims=(), collapsed_slice_dims=(0,), start_index_map=(0,))


def _shuffle(v, p):
    return lax.gather(
        v, p[:, None], dimension_numbers=_GDN, slice_sizes=(1,),
        mode=lax.GatherScatterMode.PROMISE_IN_BOUNDS)


def _lane_sum(v):
    lanes = lax.iota(jnp.int32, 16)
    for k in range(4):
        v = v + _shuffle(v, lanes ^ (1 << k))
    return v


def _rsqrt16(x):
    i = lax.bitcast_convert_type(x, jnp.int32)
    i = 0x5F3759DF - lax.shift_right_logical(i, 1)
    y = lax.bitcast_convert_type(i, jnp.float32)
    for _ in range(3):
        y = y * (1.5 - 0.5 * x * y * y)
    return y


@functools.partial(
    pl.kernel,
    out_type=jax.ShapeDtypeStruct((NROWS, H), jnp.float32),
    mesh=_mesh,
    scratch_types=[
        pltpu.VMEM((IDX_ROWS, SUB), jnp.int32),
        pltpu.VMEM((T, H), jnp.float32),
        pltpu.VMEM((H,), jnp.float32),
        pltpu.VMEM((H,), jnp.float32),
        pltpu.VMEM((CHUNK, H), jnp.float32),
        pltpu.SemaphoreType.DMA,
    ],
)
def _emb_ln_kernel(x_hbm, table_hbm, pos_hbm, gamma_hbm, beta_hbm, out_hbm,
                   idx_v, pos_v, gamma_v, beta_v, rows_v, sem):
    wid = lax.axis_index("s") * NC + lax.axis_index("c")
    base = wid * RPW

    pltpu.sync_copy(x_hbm.at[pl.ds(wid * IDX_ROWS, IDX_ROWS)], idx_v)
    pltpu.sync_copy(pos_hbm, pos_v)
    pltpu.sync_copy(gamma_hbm, gamma_v)
    pltpu.sync_copy(beta_hbm, beta_v)

    def chunk_body(j, carry):
        cps = []
        for k in range(CHUNK // SUB):
            cps.append(pltpu.async_copy(
                table_hbm.at[idx_v.at[j * (CHUNK // SUB) + k]],
                rows_v.at[pl.ds(k * SUB, SUB)],
                sem,
            ))
        for cp in cps:
            cp.wait()

        def row_body(r, rcarry):
            vs = []
            acc = None
            acc2 = None
            for i in range(HL):
                v = rows_v[r, pl.ds(16 * i, 16)] + pos_v[r, pl.ds(16 * i, 16)]
                vs.append(v)
                acc = v if acc is None else acc + v
                acc2 = v * v if acc2 is None else acc2 + v * v
            meanv = _lane_sum(acc) * (1.0 / H)
            var = _lane_sum(acc2) * (1.0 / H) - meanv * meanv
            inv = _rsqrt16(var + EPS)
            for i in range(HL):
                g = gamma_v[pl.ds(16 * i, 16)]
                bb = beta_v[pl.ds(16 * i, 16)]
                rows_v[r, pl.ds(16 * i, 16)] = (vs[i] - meanv) * (inv * g) + bb
            return rcarry

        lax.fori_loop(0, CHUNK, row_body, 0)
        pltpu.sync_copy(rows_v, out_hbm.at[pl.ds(base + j * CHUNK, CHUNK)])
        return carry

    lax.fori_loop(0, NCHUNK, chunk_body, 0)


def kernel(x, table, pos_table, gamma, beta):
    x2 = x.astype(jnp.int32).reshape(NROWS // SUB, SUB)
    pos_in = pos_table[1:T + 1]
    out = _emb_ln_kernel(x2, table, pos_in, gamma, beta)
    return out.reshape(B, T, H)


# speedup vs baseline: 3.4203x; 2.3829x over previous
import functools

import jax
import jax.numpy as jnp
import numpy as np
from jax import lax
from jax.experimental import pallas as pl
from jax.experimental.pallas import tpu as pltpu
from jax.experimental.pallas import tpu_sc as plsc

V = 100000
H = 128
B = 1024
T = 200
EPS = 1e-5

NC = 2
NS = 16
NW = NC * NS
NROWS = B * T
RPW = NROWS // NW
SUB = 100
CHUNK = 200
NCHUNK = RPW // CHUNK
IDX_ROWS = RPW // SUB
HL = H // 16
UNROLL = 4

_mesh = plsc.VectorSubcoreMesh(core_axis_name="c", subcore_axis_name="s")


_GDN = lax.GatherDimensionNumbers(
    offset_dims=(), collapsed_slice_dims=(0,), start_index_map=(0,))


def _shuffle(v, p):
    return lax.gather(
        v, p[:, None], dimension_numbers=_GDN, slice_sizes=(1,),
        mode=lax.GatherScatterMode.PROMISE_IN_BOUNDS)


def _lane_sum(v):
    lanes = lax.iota(jnp.int32, 16)
    for k in range(4):
        v = v + _shuffle(v, lanes ^ (1 << k))
    return v


def _rsqrt16(x):
    i = lax.bitcast_convert_type(x, jnp.int32)
    i = 0x5F3759DF - lax.shift_right_logical(i, 1)
    y = lax.bitcast_convert_type(i, jnp.float32)
    for _ in range(3):
        y = y * (1.5 - 0.5 * x * y * y)
    return y


@functools.partial(
    pl.kernel,
    out_type=jax.ShapeDtypeStruct((NROWS, H), jnp.float32),
    mesh=_mesh,
    scratch_types=[
        pltpu.VMEM((IDX_ROWS, SUB), jnp.int32),
        pltpu.VMEM((T, H), jnp.float32),
        pltpu.VMEM((H,), jnp.float32),
        pltpu.VMEM((H,), jnp.float32),
        pltpu.VMEM((CHUNK, H), jnp.float32),
        pltpu.SemaphoreType.DMA,
    ],
)
def _emb_ln_kernel(x_hbm, table_hbm, pos_hbm, gamma_hbm, beta_hbm, out_hbm,
                   idx_v, pos_v, gamma_v, beta_v, rows_v, sem):
    wid = lax.axis_index("s") * NC + lax.axis_index("c")
    base = wid * RPW

    pltpu.sync_copy(x_hbm.at[pl.ds(wid * IDX_ROWS, IDX_ROWS)], idx_v)
    pltpu.sync_copy(pos_hbm, pos_v)
    pltpu.sync_copy(gamma_hbm, gamma_v)
    pltpu.sync_copy(beta_hbm, beta_v)

    g_vs = [gamma_v[pl.ds(16 * i, 16)] for i in range(HL)]
    b_vs = [beta_v[pl.ds(16 * i, 16)] for i in range(HL)]

    def chunk_body(j, carry):
        cps = []
        for k in range(CHUNK // SUB):
            cps.append(pltpu.async_copy(
                table_hbm.at[idx_v.at[j * (CHUNK // SUB) + k]],
                rows_v.at[pl.ds(k * SUB, SUB)],
                sem,
            ))
        for cp in cps:
            cp.wait()

        def row_body(rr, rcarry):
            for u in range(UNROLL):
                r = rr * UNROLL + u
                vs = []
                acc = None
                acc2 = None
                for i in range(HL):
                    v = rows_v[r, pl.ds(16 * i, 16)] + pos_v[r, pl.ds(16 * i, 16)]
                    vs.append(v)
                    acc = v if acc is None else acc + v
                    acc2 = v * v if acc2 is None else acc2 + v * v
                meanv = _lane_sum(acc) * (1.0 / H)
                var = _lane_sum(acc2) * (1.0 / H) - meanv * meanv
                inv = _rsqrt16(var + EPS)
                for i in range(HL):
                    rows_v[r, pl.ds(16 * i, 16)] = (
                        (vs[i] - meanv) * (inv * g_vs[i]) + b_vs[i])
            return rcarry

        lax.fori_loop(0, CHUNK // UNROLL, row_body, 0)
        pltpu.sync_copy(rows_v, out_hbm.at[pl.ds(base + j * CHUNK, CHUNK)])
        return carry

    lax.fori_loop(0, NCHUNK, chunk_body, 0)


def kernel(x, table, pos_table, gamma, beta):
    x2 = x.astype(jnp.int32).reshape(NROWS // SUB, SUB)
    pos_in = pos_table[1:T + 1]
    out = _emb_ln_kernel(x2, table, pos_in, gamma, beta)
    return out.reshape(B, T, H)
